# baseline (speedup 1.0000x reference)
import dataclasses
import functools

import jax
import jax.numpy as jnp
from jax import lax
from jax.experimental import pallas as pl
from jax.experimental.pallas import tpu as pltpu
from jax.experimental.pallas import tpu_sc as plsc

B = 4096
L = 200
D = 32
V = 1000000
NW = 32
BPW = B // NW
IPW = BPW * L

TC_BLK = 65536
TC_GRID = -(-V // TC_BLK)
VP = TC_GRID * TC_BLK


def _tw_kernel(w_ref, t_ref, o_ref):
    r = jax.lax.dot_general(
        w_ref[...], t_ref[...],
        dimension_numbers=(((1,), (0,)), ((), ())),
        preferred_element_type=jnp.float32)
    o_ref[...] = r.reshape(TC_BLK)


def _table_times_w(table_t, w_row):
    return pl.pallas_call(
        _tw_kernel,
        grid=(TC_GRID,),
        in_specs=[
            pl.BlockSpec((1, D), lambda i: (0, 0)),
            pl.BlockSpec((D, TC_BLK), lambda i: (0, i)),
        ],
        out_specs=pl.BlockSpec((TC_BLK,), lambda i: (i,)),
        out_shape=jax.ShapeDtypeStruct((VP,), jnp.float32),
    )(w_row, table_t)


def _sc_kernel(xt_hbm, tw_hbm, bias_hbm, out_hbm, idx_v, val_v, out_v,
               bias_v, sem):
    wid = lax.axis_index("s") * 2 + lax.axis_index("c")

    pltpu.sync_copy(bias_hbm, bias_v)
    pltpu.sync_copy(
        xt_hbm.at[pl.ds(0, L), pl.ds(wid * BPW, BPW)], idx_v)

    @pl.loop(0, L)
    def _fire(l):
        pltpu.make_async_copy(
            tw_hbm.at[idx_v.at[l]],
            val_v.at[pl.ds(l * BPW, BPW)], sem).start()

    pltpu.make_async_copy(tw_hbm.at[pl.ds(0, IPW)], val_v, sem).wait()

    biasv = bias_v[pl.ds(0, 16)]
    scale = 1.0 / L
    zero = jnp.zeros((16,), jnp.float32)

    def acc_body(l, accs):
        base = l * BPW
        return tuple(a + val_v[pl.ds(base + g * 16, 16)]
                     for g, a in enumerate(accs))

    accs = lax.fori_loop(0, L, acc_body, (zero,) * (BPW // 16), unroll=4)
    for g, a in enumerate(accs):
        out_v[pl.ds(g * 16, 16)] = a * scale + biasv

    pltpu.sync_copy(out_v, out_hbm.at[pl.ds(wid * BPW, BPW)])


def kernel(x, table, W, b):
    xt = x.T.astype(jnp.int32)
    w_row = W.reshape(1, D).astype(jnp.float32)
    tw = _table_times_w(table.T, w_row)
    bias16 = jnp.broadcast_to(b.astype(jnp.float32), (16,))

    mesh = plsc.VectorSubcoreMesh(core_axis_name="c", subcore_axis_name="s")
    cp = pltpu.CompilerParams()
    fields = pltpu.CompilerParams.__dataclass_fields__
    if "needs_layout_passes" in fields:
        cp = dataclasses.replace(cp, needs_layout_passes=False)
    if "use_tc_tiling_on_sc" in fields:
        cp = dataclasses.replace(cp, use_tc_tiling_on_sc=True)
    run = functools.partial(
        pl.kernel,
        compiler_params=cp,
        out_type=jax.ShapeDtypeStruct((B,), jnp.float32),
        mesh=mesh,
        scratch_types=[
            pltpu.VMEM((L, BPW), jnp.int32),
            pltpu.VMEM((IPW,), jnp.float32),
            pltpu.VMEM((BPW,), jnp.float32),
            pltpu.VMEM((16,), jnp.float32),
            pltpu.SemaphoreType.DMA,
        ],
    )(_sc_kernel)

    out = run(xt, tw, bias16)
    return out.reshape(B, 1)


# speedup vs baseline: 1.0006x; 1.0006x over previous
import dataclasses
import functools

import jax
import jax.numpy as jnp
from jax import lax
from jax.experimental import pallas as pl
from jax.experimental.pallas import tpu as pltpu
from jax.experimental.pallas import tpu_sc as plsc

B = 4096
L = 200
D = 32
V = 1000000
NW = 32
BPW = B // NW
IPW = BPW * L

TC_BLK = 65536
TC_GRID = -(-V // TC_BLK)
VP = TC_GRID * TC_BLK


def _tw_kernel(w_ref, t_ref, o_ref):
    r = jax.lax.dot_general(
        w_ref[...], t_ref[...],
        dimension_numbers=(((1,), (0,)), ((), ())),
        preferred_element_type=jnp.float32)
    o_ref[...] = r.reshape(TC_BLK)


def _table_times_w(table_t, w_row):
    return pl.pallas_call(
        _tw_kernel,
        grid=(TC_GRID,),
        in_specs=[
            pl.BlockSpec((1, D), lambda i: (0, 0)),
            pl.BlockSpec((D, TC_BLK), lambda i: (0, i)),
        ],
        out_specs=pl.BlockSpec((TC_BLK,), lambda i: (i,)),
        out_shape=jax.ShapeDtypeStruct((VP,), jnp.float32),
    )(w_row, table_t)


def _sc_kernel(xt_hbm, tw_hbm, bias_hbm, out_hbm, idx_v, val_v, out_v,
               bias_v, sem, ssem):
    wid = lax.axis_index("s") * 2 + lax.axis_index("c")

    pltpu.sync_copy(bias_hbm, bias_v)
    NST = 5
    LC = L // NST
    for s in range(NST):
        pltpu.make_async_copy(
            xt_hbm.at[pl.ds(s * LC, LC), pl.ds(wid * BPW, BPW)],
            idx_v.at[pl.ds(s * LC, LC)], ssem.at[s]).start()

    for s in range(NST):
        pltpu.make_async_copy(
            xt_hbm.at[pl.ds(s * LC, LC), pl.ds(wid * BPW, BPW)],
            idx_v.at[pl.ds(s * LC, LC)], ssem.at[s]).wait()

        @pl.loop(s * LC, (s + 1) * LC)
        def _fire(l):
            pltpu.make_async_copy(
                tw_hbm.at[idx_v.at[l]],
                val_v.at[pl.ds(l * BPW, BPW)], sem).start()

    pltpu.make_async_copy(tw_hbm.at[pl.ds(0, IPW)], val_v, sem).wait()

    biasv = bias_v[pl.ds(0, 16)]
    scale = 1.0 / L
    zero = jnp.zeros((16,), jnp.float32)

    def acc_body(l, accs):
        base = l * BPW
        return tuple(a + val_v[pl.ds(base + g * 16, 16)]
                     for g, a in enumerate(accs))

    accs = lax.fori_loop(0, L, acc_body, (zero,) * (BPW // 16), unroll=4)
    for g, a in enumerate(accs):
        out_v[pl.ds(g * 16, 16)] = a * scale + biasv

    pltpu.sync_copy(out_v, out_hbm.at[pl.ds(wid * BPW, BPW)])


def kernel(x, table, W, b):
    xt = x.T.astype(jnp.int32)
    w_row = W.reshape(1, D).astype(jnp.float32)
    tw = _table_times_w(table.T, w_row)
    bias16 = jnp.broadcast_to(b.astype(jnp.float32), (16,))

    mesh = plsc.VectorSubcoreMesh(core_axis_name="c", subcore_axis_name="s")
    cp = pltpu.CompilerParams()
    fields = pltpu.CompilerParams.__dataclass_fields__
    if "needs_layout_passes" in fields:
        cp = dataclasses.replace(cp, needs_layout_passes=False)
    if "use_tc_tiling_on_sc" in fields:
        cp = dataclasses.replace(cp, use_tc_tiling_on_sc=True)
    run = functools.partial(
        pl.kernel,
        compiler_params=cp,
        out_type=jax.ShapeDtypeStruct((B,), jnp.float32),
        mesh=mesh,
        scratch_types=[
            pltpu.VMEM((L, BPW), jnp.int32),
            pltpu.VMEM((IPW,), jnp.float32),
            pltpu.VMEM((BPW,), jnp.float32),
            pltpu.VMEM((16,), jnp.float32),
            pltpu.SemaphoreType.DMA,
            pltpu.SemaphoreType.DMA((5,)),
        ],
    )(_sc_kernel)

    out = run(xt, tw, bias16)
    return out.reshape(B, 1)


# speedup vs baseline: 1.0014x; 1.0008x over previous
import dataclasses
import functools

import jax
import jax.numpy as jnp
from jax import lax
from jax.experimental import pallas as pl
from jax.experimental.pallas import tpu as pltpu
from jax.experimental.pallas import tpu_sc as plsc

B = 4096
L = 200
D = 32
V = 1000000
NW = 32
BPW = B // NW
IPW = BPW * L

TC_BLK = 65536
TC_GRID = -(-V // TC_BLK)
VP = TC_GRID * TC_BLK


def _tw_kernel(w_ref, t_ref, o_ref):
    r = jax.lax.dot_general(
        w_ref[...], t_ref[...],
        dimension_numbers=(((1,), (0,)), ((), ())),
        preferred_element_type=jnp.float32)
    o_ref[...] = r.reshape(TC_BLK)


def _table_times_w(table_t, w_row):
    return pl.pallas_call(
        _tw_kernel,
        grid=(TC_GRID,),
        in_specs=[
            pl.BlockSpec((1, D), lambda i: (0, 0)),
            pl.BlockSpec((D, TC_BLK), lambda i: (0, i)),
        ],
        out_specs=pl.BlockSpec((TC_BLK,), lambda i: (i,)),
        out_shape=jax.ShapeDtypeStruct((VP,), jnp.float32),
    )(w_row, table_t)


def _sc_kernel(xt_hbm, tw_hbm, bias_hbm, out_hbm, idx_v, val_v, out_v,
               bias_v, sem):
    wid = lax.axis_index("s") * 2 + lax.axis_index("c")

    pltpu.sync_copy(bias_hbm, bias_v)
    pltpu.sync_copy(
        xt_hbm.at[pl.ds(0, L), pl.ds(wid * BPW, BPW)], idx_v)

    @pl.loop(0, L)
    def _fire(l):
        pltpu.make_async_copy(
            tw_hbm.at[idx_v.at[l]],
            val_v.at[pl.ds(l * BPW, BPW)], sem).start()

    pltpu.make_async_copy(tw_hbm.at[pl.ds(0, IPW)], val_v, sem).wait()

    biasv = bias_v[pl.ds(0, 16)]
    scale = 1.0 / L
    zero = jnp.zeros((16,), jnp.float32)

    def acc_body(l, accs):
        base = l * BPW
        return tuple(a + val_v[pl.ds(base + g * 16, 16)]
                     for g, a in enumerate(accs))

    accs = lax.fori_loop(0, L, acc_body, (zero,) * (BPW // 16), unroll=4)
    for g, a in enumerate(accs):
        out_v[pl.ds(g * 16, 16)] = a * scale + biasv

    pltpu.sync_copy(out_v, out_hbm.at[pl.ds(wid * BPW, BPW)])


def kernel(x, table, W, b):
    xt = x.T.astype(jnp.int32)
    w_row = W.reshape(1, D).astype(jnp.float32)
    tw = _table_times_w(table.T, w_row)
    bias16 = jnp.broadcast_to(b.astype(jnp.float32), (16,))

    mesh = plsc.VectorSubcoreMesh(core_axis_name="c", subcore_axis_name="s")
    cp = pltpu.CompilerParams()
    fields = pltpu.CompilerParams.__dataclass_fields__
    if "needs_layout_passes" in fields:
        cp = dataclasses.replace(cp, needs_layout_passes=False)
    if "use_tc_tiling_on_sc" in fields:
        cp = dataclasses.replace(cp, use_tc_tiling_on_sc=True)
    run = functools.partial(
        pl.kernel,
        compiler_params=cp,
        out_type=jax.ShapeDtypeStruct((B,), jnp.float32),
        mesh=mesh,
        scratch_types=[
            pltpu.VMEM((L, BPW), jnp.int32),
            pltpu.VMEM((IPW,), jnp.float32),
            pltpu.VMEM((BPW,), jnp.float32),
            pltpu.VMEM((16,), jnp.float32),
            pltpu.SemaphoreType.DMA,
        ],
    )(_sc_kernel)

    out = run(xt, tw, bias16)
    return out.reshape(B, 1)
